# baseline (speedup 1.0000x reference)
import functools

import jax
import jax.numpy as jnp
from jax import lax
from jax.experimental import pallas as pl
from jax.experimental.pallas import tpu as pltpu
from jax.experimental.pallas import tpu_sc as plsc

NUM_USERS = 1000000
NUM_ITEMS = 1000000
EMBED_K = 16
BATCH = 16384

_NC = 2
_NS = 16
_NW = _NC * _NS
_BPW = BATCH // _NW
_L = 16


def _body(uidx_hbm, vidx_hbm, w_hbm, h_hbm, out_hbm,
          uidx_v, vidx_v, urows_v, vrows_v, out_v, sem_u, sem_v):
    wid = lax.axis_index("s") * _NC + lax.axis_index("c")
    base = wid * _BPW

    pltpu.sync_copy(uidx_hbm.at[pl.ds(base, _BPW)], uidx_v)
    pltpu.sync_copy(vidx_hbm.at[pl.ds(base, _BPW)], vidx_v)

    cp_u = pltpu.make_async_copy(w_hbm.at[uidx_v], urows_v, sem_u)
    cp_v = pltpu.make_async_copy(h_hbm.at[vidx_v], vrows_v, sem_v)
    cp_u.start()
    cp_v.start()
    cp_u.wait()
    cp_v.wait()

    lanes = lax.iota(jnp.int32, _L)

    def group(g, _):
        z = jnp.zeros((_L,), jnp.float32)
        for j in range(_L):
            i = g * _L + j
            u = urows_v[i, :]
            v = vrows_v[i, :]
            d = jnp.sum(u * v)
            z = jnp.where(lanes == j, d, z)
        inner = 1.0 / (1.0 + jnp.exp(-z))
        pred = 1.0 / (1.0 + jnp.exp(-inner))
        out_v[pl.ds(g * _L, _L)] = pred
        return 0

    lax.fori_loop(0, _BPW // _L, group, 0)

    pltpu.sync_copy(out_v, out_hbm.at[pl.ds(base, _BPW)])


@jax.jit
def _run(uidx, vidx, w, h):
    mesh = plsc.VectorSubcoreMesh(core_axis_name="c", subcore_axis_name="s")
    f = pl.kernel(
        _body,
        mesh=mesh,
        out_type=jax.ShapeDtypeStruct((BATCH,), jnp.float32),
        compiler_params=pltpu.CompilerParams(
            needs_layout_passes=False, use_tc_tiling_on_sc=False),
        scratch_types=[
            pltpu.VMEM((_BPW,), jnp.int32),
            pltpu.VMEM((_BPW,), jnp.int32),
            pltpu.VMEM((_BPW, EMBED_K), jnp.float32),
            pltpu.VMEM((_BPW, EMBED_K), jnp.float32),
            pltpu.VMEM((_BPW,), jnp.float32),
            pltpu.SemaphoreType.DMA,
            pltpu.SemaphoreType.DMA,
        ],
    )
    return f(uidx, vidx, w, h)


def kernel(x, W, H):
    uidx = x[:, 0]
    vidx = x[:, 1]
    return _run(uidx, vidx, W, H)
